# baseline (speedup 1.0000x reference)
import jax
import jax.numpy as jnp
from jax import lax
from jax.experimental import pallas as pl
from jax.experimental.pallas import tpu as pltpu
from jax.experimental.pallas import tpu_sc as plsc

L = 16
MW = 32


def _build(T, D, B, C, NC, NS):
    NW = NC * NS
    PW = B // NW
    K = PW // C
    D2 = D // 2
    TP = T + 2
    TD = TP * D2
    mesh = plsc.VectorSubcoreMesh(core_axis_name="c", subcore_axis_name="s")

    def body(ute, ite, umi, imi, user, item, out,
             uidx, iidx, uemb, iemb, umis, imis, outv, sem_a, sem_b):
        wid = lax.axis_index("s") * NC + lax.axis_index("c")
        base = wid * PW

        for k in range(K):
            pltpu.sync_copy(user.at[pl.ds(base + k * C, C)], uidx.at[k])
            pltpu.sync_copy(item.at[pl.ds(base + k * C, C)], iidx.at[k])

        lane = lax.iota(jnp.int32, L)
        t0c = lane * D2
        t1c = jnp.minimum((lane + L) * D2, (T - 1) * D2)
        tb0 = lane
        tb1 = jnp.minimum(lane + L, T - 1)
        hi_mask = lane < (T - L)
        lane0 = lane == 0

        sems = [sem_a, sem_b]
        tabs = [ute, ite, umi, imi]
        idxs = [uidx, iidx, uidx, iidx]
        bufs = [uemb, iemb, umis, imis]
        pending = {}

        def start(k):
            s = k % 2
            pending[s] = [
                pltpu.async_copy(tab.at[idx.at[k]], buf.at[s], sems[s])
                for tab, idx, buf in zip(tabs, idxs, bufs)
            ]

        def drain(k):
            for h in pending[k % 2]:
                h.wait()

        def compute(k):
            s = k % 2
            ue, ie = uemb.at[s], iemb.at[s]
            um_, im_ = umis.at[s], imis.at[s]
            zero = jnp.zeros((L,), jnp.float32)

            def unpk(w):
                return plsc.unpack(
                    plsc.bitcast(w, jnp.bfloat16),
                    format=plsc.PackFormat.INTERLEAVED,
                    preferred_element_type=jnp.float32)

            def elem(b, carry):
                row = jnp.full((L,), b, jnp.int32)
                s0 = zero
                s1 = zero
                for d in range(D2):
                    c0 = t0c + d
                    c1 = t1c + d
                    ua0, ub0 = unpk(plsc.load_gather(ue, [row, c0]))
                    va0, vb0 = unpk(plsc.load_gather(ie, [row, c0]))
                    ua1, ub1 = unpk(plsc.load_gather(ue, [row, c1]))
                    va1, vb1 = unpk(plsc.load_gather(ie, [row, c1]))
                    s0 = s0 + ua0 * va0 + ub0 * vb0
                    s1 = s1 + ua1 * va1 + ub1 * vb1
                ub0, uc0 = unpk(plsc.load_gather(um_, [row, tb0]))
                ib0, ic0 = unpk(plsc.load_gather(im_, [row, tb0]))
                ub1, uc1 = unpk(plsc.load_gather(um_, [row, tb1]))
                ib1, ic1 = unpk(plsc.load_gather(im_, [row, tb1]))
                sc0 = s0 + ub0 + ib0
                sc1 = s1 + ub1 + ib1
                sg0 = 1.0 / (1.0 + jnp.exp(-sc0))
                sg1 = 1.0 / (1.0 + jnp.exp(-sc1))
                co0 = uc0 + ic0
                co1 = jnp.where(hi_mask, uc1 + ic1, 0.0)
                p = sg0 * co0 + sg1 * co1
                rating = jnp.full((L,), jnp.sum(p), jnp.float32)
                plsc.store_scatter(outv, [row + k * C], rating, mask=lane0)
                return carry

            lax.fori_loop(0, C, elem, 0)

        start(0)
        for k in range(K):
            if k + 1 < K:
                start(k + 1)
            drain(k)
            compute(k)
        pltpu.sync_copy(outv, out.at[pl.ds(base, PW)])

    return pl.kernel(
        body,
        out_type=jax.ShapeDtypeStruct((B,), jnp.float32),
        mesh=mesh,
        scratch_types=[
            pltpu.VMEM((K, C), jnp.int32),
            pltpu.VMEM((K, C), jnp.int32),
            pltpu.VMEM((2, C, TD), jnp.float32),
            pltpu.VMEM((2, C, TD), jnp.float32),
            pltpu.VMEM((2, C, MW), jnp.float32),
            pltpu.VMEM((2, C, MW), jnp.float32),
            pltpu.VMEM((PW,), jnp.float32),
            pltpu.SemaphoreType.DMA,
            pltpu.SemaphoreType.DMA,
        ],
        compiler_params=pltpu.CompilerParams(
            needs_layout_passes=False, use_tc_tiling_on_sc=False),
    )


def _bf16_bits(x):
    return jax.lax.bitcast_convert_type(
        x.astype(jnp.bfloat16), jnp.uint16).astype(jnp.uint32)


def _pack_misc(bias, coeff, gbias, gcoeff, T):
    w = _bf16_bits(bias + gbias) | (_bf16_bits(coeff + gcoeff) << 16)
    w = jnp.pad(w, ((0, 0), (0, MW - T)))
    return jax.lax.bitcast_convert_type(w, jnp.float32)


def _pack_emb(emb, n, T, D):
    e = emb.astype(jnp.bfloat16)
    lo = jax.lax.bitcast_convert_type(e[:, :, 0::2], jnp.uint16).astype(jnp.uint32)
    hi = jax.lax.bitcast_convert_type(e[:, :, 1::2], jnp.uint16).astype(jnp.uint32)
    w = jnp.pad(lo | (hi << 16), ((0, 0), (0, 2), (0, 0)))
    return jax.lax.bitcast_convert_type(w, jnp.float32).reshape(n, (T + 2) * D // 2)


def kernel(user_tag_embeddings, item_tag_embeddings, user_aspect_bias,
           item_aspect_bias, global_aspect_bias, user_coeff, item_coeff,
           global_coeff, user, item):
    U, T, D = user_tag_embeddings.shape
    I = item_tag_embeddings.shape[0]
    B = user.shape[0]
    info = plsc.get_sparse_core_info()
    fn = _build(T, D, B, 64, info.num_cores, info.num_subcores)
    umi = _pack_misc(user_aspect_bias, user_coeff,
                     global_aspect_bias, global_coeff, T)
    imi = _pack_misc(item_aspect_bias, item_coeff,
                     jnp.zeros((1, T), jnp.float32),
                     jnp.zeros((1, T), jnp.float32), T)
    return fn(
        _pack_emb(user_tag_embeddings, U, T, D),
        _pack_emb(item_tag_embeddings, I, T, D),
        umi,
        imi,
        user.astype(jnp.int32),
        item.astype(jnp.int32),
    )


# speedup vs baseline: 1.2504x; 1.2504x over previous
import jax
import jax.numpy as jnp
from jax import lax
from jax.experimental import pallas as pl
from jax.experimental.pallas import tpu as pltpu
from jax.experimental.pallas import tpu_sc as plsc

L = 16
MW = 32


def _build(T, D, B, C, NC, NS):
    NW = NC * NS
    PW = B // NW
    K = PW // C
    TD = T * D
    mesh = plsc.VectorSubcoreMesh(core_axis_name="c", subcore_axis_name="s")

    def body(ute, ite, umi, imi, user, item, out,
             uidx, iidx, uemb, iemb, umis, imis, outv, sem_a, sem_b):
        wid = lax.axis_index("s") * NC + lax.axis_index("c")
        base = wid * PW

        for k in range(K):
            pltpu.sync_copy(user.at[pl.ds(base + k * C, C)], uidx.at[k])
            pltpu.sync_copy(item.at[pl.ds(base + k * C, C)], iidx.at[k])

        lane = lax.iota(jnp.int32, L)
        t0c = lane * D
        t1c = jnp.minimum((lane + L) * D, (T - 1) * D)
        tb0 = lane
        tb1 = jnp.minimum(lane + L, T - 1)
        hi_mask = lane < (T - L)
        lane0 = lane == 0

        sems = [sem_a, sem_b]
        tabs = [ute, ite, umi, imi]
        idxs = [uidx, iidx, uidx, iidx]
        bufs = [uemb, iemb, umis, imis]
        pending = {}

        def start(k):
            s = k % 2
            pending[s] = [
                pltpu.async_copy(tab.at[idx.at[k]], buf.at[s], sems[s])
                for tab, idx, buf in zip(tabs, idxs, bufs)
            ]

        def drain(k):
            for h in pending[k % 2]:
                h.wait()

        def unpk(w):
            return plsc.unpack(
                plsc.bitcast(w, jnp.bfloat16),
                format=plsc.PackFormat.INTERLEAVED,
                preferred_element_type=jnp.float32)

        def compute(k):
            s = k % 2
            ue, ie = uemb.at[s], iemb.at[s]
            um_, im_ = umis.at[s], imis.at[s]
            zero = jnp.zeros((L,), jnp.float32)

            def elem(b, carry):
                row = jnp.full((L,), b, jnp.int32)
                s0 = zero
                s1 = zero
                for d in range(D):
                    c0 = t0c + d
                    c1 = t1c + d
                    u0 = plsc.load_gather(ue, [row, c0])
                    v0 = plsc.load_gather(ie, [row, c0])
                    u1 = plsc.load_gather(ue, [row, c1])
                    v1 = plsc.load_gather(ie, [row, c1])
                    s0 = s0 + u0 * v0
                    s1 = s1 + u1 * v1
                ub0, uc0 = unpk(plsc.load_gather(um_, [row, tb0]))
                ib0, ic0 = unpk(plsc.load_gather(im_, [row, tb0]))
                ub1, uc1 = unpk(plsc.load_gather(um_, [row, tb1]))
                ib1, ic1 = unpk(plsc.load_gather(im_, [row, tb1]))
                sc0 = s0 + ub0 + ib0
                sc1 = s1 + ub1 + ib1
                sg0 = 1.0 / (1.0 + jnp.exp(-sc0))
                sg1 = 1.0 / (1.0 + jnp.exp(-sc1))
                co0 = uc0 + ic0
                co1 = jnp.where(hi_mask, uc1 + ic1, 0.0)
                p = sg0 * co0 + sg1 * co1
                rating = jnp.full((L,), jnp.sum(p), jnp.float32)
                plsc.store_scatter(outv, [row + k * C], rating, mask=lane0)
                return carry

            lax.fori_loop(0, C, elem, 0)

        start(0)
        for k in range(K):
            if k + 1 < K:
                start(k + 1)
            drain(k)
            compute(k)
        pltpu.sync_copy(outv, out.at[pl.ds(base, PW)])

    return pl.kernel(
        body,
        out_type=jax.ShapeDtypeStruct((B,), jnp.float32),
        mesh=mesh,
        scratch_types=[
            pltpu.VMEM((K, C), jnp.int32),
            pltpu.VMEM((K, C), jnp.int32),
            pltpu.VMEM((2, C, TD), jnp.float32),
            pltpu.VMEM((2, C, TD), jnp.float32),
            pltpu.VMEM((2, C, MW), jnp.float32),
            pltpu.VMEM((2, C, MW), jnp.float32),
            pltpu.VMEM((PW,), jnp.float32),
            pltpu.SemaphoreType.DMA,
            pltpu.SemaphoreType.DMA,
        ],
        compiler_params=pltpu.CompilerParams(
            needs_layout_passes=False, use_tc_tiling_on_sc=False),
    )


def _bf16_bits(x):
    return jax.lax.bitcast_convert_type(
        x.astype(jnp.bfloat16), jnp.uint16).astype(jnp.uint32)


def _pack_misc(bias, coeff, gbias, gcoeff, T):
    w = _bf16_bits(bias + gbias) | (_bf16_bits(coeff + gcoeff) << 16)
    w = jnp.pad(w, ((0, 0), (0, MW - T)))
    return jax.lax.bitcast_convert_type(w, jnp.float32)


def kernel(user_tag_embeddings, item_tag_embeddings, user_aspect_bias,
           item_aspect_bias, global_aspect_bias, user_coeff, item_coeff,
           global_coeff, user, item):
    U, T, D = user_tag_embeddings.shape
    I = item_tag_embeddings.shape[0]
    B = user.shape[0]
    info = plsc.get_sparse_core_info()
    fn = _build(T, D, B, 64, info.num_cores, info.num_subcores)
    umi = _pack_misc(user_aspect_bias, user_coeff,
                     global_aspect_bias, global_coeff, T)
    imi = _pack_misc(item_aspect_bias, item_coeff,
                     jnp.zeros((1, T), jnp.float32),
                     jnp.zeros((1, T), jnp.float32), T)
    return fn(
        user_tag_embeddings.reshape(U, T * D),
        item_tag_embeddings.reshape(I, T * D),
        umi,
        imi,
        user.astype(jnp.int32),
        item.astype(jnp.int32),
    )
